# baseline (speedup 1.0000x reference)
import jax
import jax.numpy as jnp
from jax import lax
from jax.experimental import pallas as pl
from jax.experimental.pallas import tpu as pltpu
from jax.experimental.pallas import tpu_sc as plsc

BSZ, BEAM, VOCAB = 64, 4, 100000
N = BEAM * VOCAB
K = 8
L = 16

NW = 32
ROWS_PER_W = BSZ // NW

RV = 25
GELEM = RV * L
NGROUP = N // GELEM
NGPAD = 1008
NSUPER = NGPAD // 16
GP_BEAM = VOCAB // GELEM

BLK = 20000
NBLK = N // BLK
GPB = BLK // GELEM

NEG = -3.0e38
IMAX = 2147483647


def _sc_body(rows_per_w,
             lp_hbm, bias_hbm, vals_hbm, flats_hbm,
             buf0, buf1, a1, rescan, idxbuf, biasv, ovst, ofst,
             sem0, sem1, sem2):
    wid = lax.axis_index("c") * 16 + lax.axis_index("s")
    iota = lax.iota(jnp.int32, L)
    negv = jnp.full((L,), NEG, dtype=jnp.float32)
    zerov = jnp.zeros((L,), dtype=jnp.int32)

    for g in range(NGROUP, NGPAD):
        a1[pl.ds(g * L, L)] = negv

    def row_body(r, _):
        row = wid * rows_per_w + r
        pltpu.sync_copy(bias_hbm.at[row], biasv)

        bufs = (buf0, buf1)
        sems = (sem0, sem1)

        def start(t):
            return pltpu.async_copy(
                lp_hbm.at[row, pl.ds(t * BLK, BLK)], bufs[t % 2], sems[t % 2])

        copies = [start(0), None]
        for t in range(NBLK):
            if t + 1 < NBLK:
                copies[(t + 1) % 2] = start(t + 1)
            copies[t % 2].wait()
            buf = bufs[t % 2]
            beam = t // (NBLK // BEAM)
            bv = biasv[pl.ds(beam * L, L)]

            def grp(g, _, t=t, buf=buf, bv=bv):
                off = g * GELEM
                accs = []
                for j in range(5):
                    acc = buf[pl.ds(off + (j * 5) * L, L)] + bv
                    for u in range(1, 5):
                        acc = jnp.maximum(
                            acc, buf[pl.ds(off + (j * 5 + u) * L, L)] + bv)
                    accs.append(acc)
                m01 = jnp.maximum(accs[0], accs[1])
                m23 = jnp.maximum(accs[2], accs[3])
                am = jnp.maximum(jnp.maximum(m01, m23), accs[4])
                a1[pl.ds((t * GPB + g) * L, L)] = am
                return 0

            lax.fori_loop(0, GPB, grp, 0)

        def sup(t, mvs):
            m = a1[pl.ds(t * 16 * L, L)]
            for j in range(1, 16):
                m = jnp.maximum(m, a1[pl.ds((t * 16 + j) * L, L)])
            s = jnp.max(m)
            blk_i = t // L
            lane = t - blk_i * L
            out = []
            for j in range(4):
                out.append(jnp.where((blk_i == j) & (iota == lane), s, mvs[j]))
            return tuple(out)

        mv = list(lax.fori_loop(0, NSUPER, sup, (negv, negv, negv, negv)))

        sgs = []
        for _ in range(K):
            V, ID = mv[0], iota
            for j in range(1, 4):
                idj = iota + j * L
                gt = mv[j] > V
                V = jnp.where(gt, mv[j], V)
                ID = jnp.where(gt, idj, ID)
            gmax = jnp.max(V)
            sg = jnp.min(jnp.where(V == gmax, ID, IMAX))
            sgs.append(sg)
            mv = [jnp.where((iota + j * L) == sg, NEG, mv[j]) for j in range(4)]

        gvs, gis = [], []
        for k in range(K):
            gv = negv
            for j in range(16):
                m16 = a1[pl.ds((sgs[k] * 16 + j) * L, L)]
                gv = jnp.where(iota == j, jnp.max(m16), gv)
            gvs.append(gv)
            gis.append(sgs[k] * 16 + iota)

        gsel = []
        for _ in range(K):
            V, ID = gvs[0], gis[0]
            for j in range(1, K):
                x, idj = gvs[j], gis[j]
                gt = (x > V) | ((x == V) & (idj < ID))
                V = jnp.where(gt, x, V)
                ID = jnp.where(gt, idj, ID)
            gmax = jnp.max(V)
            gstar = jnp.min(jnp.where(V == gmax, ID, IMAX))
            gsel.append(gstar)
            gvs = [jnp.where(gis[j] == gstar, NEG, gvs[j]) for j in range(K)]

        gvec = jnp.full((L,), IMAX, dtype=jnp.int32)
        for k in range(K):
            gvec = jnp.where(iota == k, gsel[k], gvec)
        gsorted, _ = plsc.sort_key_val(gvec, gvec)
        gs = [jnp.min(jnp.where(iota == k, gsorted, IMAX)) for k in range(K)]

        rcopies = [
            pltpu.async_copy(lp_hbm.at[row, pl.ds(gs[k] * GELEM, GELEM)],
                             rescan.at[pl.ds(k * GELEM, GELEM)], sem2)
            for k in range(K)
        ]
        for c in rcopies:
            c.wait()

        for k in range(K):
            g = gs[k]
            beam = g // GP_BEAM
            bvk = biasv[pl.ds(beam * L, L)]
            basev = g * GELEM + iota

            def rbias(j, _, k=k, bvk=bvk, basev=basev):
                o = k * GELEM + j * L
                rescan[pl.ds(o, L)] = rescan[pl.ds(o, L)] + bvk
                idxbuf[pl.ds(o, L)] = basev + j * L
                return 0

            lax.fori_loop(0, RV, rbias, 0)

        ov = jnp.zeros((L,), dtype=jnp.float32)
        of = zerov
        for k in range(K):
            def ext(u, c):
                V, IX, P = c
                x = rescan[pl.ds(u * L, L)]
                ix = idxbuf[pl.ds(u * L, L)]
                gt = x > V
                return (jnp.where(gt, x, V), jnp.where(gt, ix, IX),
                        jnp.where(gt, jnp.broadcast_to(u, (L,)), P))

            V, IX, P = lax.fori_loop(0, K * RV, ext, (negv, zerov, zerov))
            gmax = jnp.max(V)
            lm = V == gmax
            istar = jnp.min(jnp.where(lm, IX, IMAX))
            wl = lm & (IX == istar)
            pos = (jnp.min(jnp.where(wl, P, IMAX)) * L
                   + jnp.min(jnp.where(wl, iota, L)))
            plsc.store_scatter(rescan, [jnp.broadcast_to(pos, (L,))], negv,
                               mask=iota == 0)
            ov = jnp.where(iota == k, gmax, ov)
            of = jnp.where(iota == k, istar, of)

        ovst[...] = ov
        ofst[...] = of
        pltpu.sync_copy(ovst, vals_hbm.at[row])
        pltpu.sync_copy(ofst, flats_hbm.at[row])
        return 0

    lax.fori_loop(0, rows_per_w, row_body, 0)


def _make_sc_topk(rows_per_w):
    import functools
    rows = NW * rows_per_w
    return pl.kernel(
        functools.partial(_sc_body, rows_per_w),
        out_type=[
            jax.ShapeDtypeStruct((rows, L), jnp.float32),
            jax.ShapeDtypeStruct((rows, L), jnp.int32),
        ],
        mesh=plsc.VectorSubcoreMesh(core_axis_name="c", subcore_axis_name="s",
                                    num_cores=2, num_subcores=16),
        compiler_params=pltpu.CompilerParams(use_tc_tiling_on_sc=False,
                                             needs_layout_passes=False),
        scratch_types=[
            pltpu.VMEM((BLK,), jnp.float32),
            pltpu.VMEM((BLK,), jnp.float32),
            pltpu.VMEM((NGPAD * L,), jnp.float32),
            pltpu.VMEM((K * GELEM,), jnp.float32),
            pltpu.VMEM((K * GELEM,), jnp.int32),
            pltpu.VMEM((BEAM * L,), jnp.float32),
            pltpu.VMEM((L,), jnp.float32),
            pltpu.VMEM((L,), jnp.int32),
            pltpu.SemaphoreType.DMA,
            pltpu.SemaphoreType.DMA,
            pltpu.SemaphoreType.DMA,
        ],
    )


NCHUNK = 2
CROWS = BSZ // NCHUNK
_sc_topk_chunk = _make_sc_topk(CROWS // NW)


def kernel(step, lprobs, scores):
    bsz, beam, vocab = lprobs.shape
    bias = jnp.take(scores, step - 1, axis=2)
    biasb = jnp.broadcast_to(bias[:, :, None], (bsz, beam, L))
    biasb = biasb.reshape(bsz, beam * L)
    outs = []
    for h in range(NCHUNK):
        lp = lprobs[h * CROWS:(h + 1) * CROWS].reshape(CROWS, beam * vocab)
        outs.append(_sc_topk_chunk(lp, biasb[h * CROWS:(h + 1) * CROWS]))
    vals = jnp.concatenate([o[0] for o in outs], axis=0)
    flats = jnp.concatenate([o[1] for o in outs], axis=0)
    vals = vals[:, :K]
    flats = flats[:, :K]
    return (vals, flats % vocab, flats // vocab)


# speedup vs baseline: 1.2137x; 1.2137x over previous
import functools

import jax
import jax.numpy as jnp
from jax import lax
from jax.experimental import pallas as pl
from jax.experimental.pallas import tpu as pltpu
from jax.experimental.pallas import tpu_sc as plsc

BSZ, BEAM, VOCAB = 64, 4, 100000
K = 8
L = 16

GELEM = 512
GP_BEAM = 196
VPAD = GP_BEAM * GELEM
NGROUP = BEAM * GP_BEAM
CAND = K * GELEM

NW = 32
ROWS_PER_W = BSZ // NW

NEG = -3.0e38
IMAX = 2147483647


def _tc_body(lp_ref, bias_ref, cand_ref, gid_ref, y2_ref):
    x = lp_ref[0]
    b = bias_ref[0]
    y2_ref[:, :VOCAB] = x + b
    y2_ref[:, VOCAB:] = jnp.full((BEAM, VPAD - VOCAB), NEG, dtype=jnp.float32)
    y2 = y2_ref[...]
    m = jnp.max(y2.reshape(BEAM, GP_BEAM, GELEM), axis=2)
    gids = (jax.lax.broadcasted_iota(jnp.int32, (BEAM, GP_BEAM), 0) * GP_BEAM
            + jax.lax.broadcasted_iota(jnp.int32, (BEAM, GP_BEAM), 1))
    rows4 = jax.lax.broadcasted_iota(jnp.int32, (BEAM, 1), 0)
    io16 = jax.lax.broadcasted_iota(jnp.int32, (1, 1, L), 2)
    gidvec = jnp.full((1, 1, L), IMAX, dtype=jnp.int32)
    for k in range(K):
        gmax = jnp.max(m)
        gid = jnp.min(jnp.where(m == gmax, gids, IMAX))
        m = jnp.where(gids == gid, NEG, m)
        beam = gid // GP_BEAM
        j = gid % GP_BEAM
        chunk4 = y2_ref[:, pl.ds(j * GELEM, GELEM)]
        chunk = jnp.max(jnp.where(rows4 == beam, chunk4, NEG), axis=0)
        cand_ref[0, k] = chunk
        gidvec = jnp.where(io16 == k, gid, gidvec)
    gid_ref[...] = gidvec


_tc_stage = None


def _make_tc_stage():
    return pl.pallas_call(
        _tc_body,
        grid=(BSZ,),
        in_specs=[
            pl.BlockSpec((1, BEAM, VOCAB), lambda i: (i, 0, 0)),
            pl.BlockSpec((1, BEAM, 1), lambda i: (i, 0, 0)),
        ],
        out_specs=[
            pl.BlockSpec((1, K, GELEM), lambda i: (i, 0, 0)),
            pl.BlockSpec((1, 1, L), lambda i: (i, 0, 0)),
        ],
        out_shape=[
            jax.ShapeDtypeStruct((BSZ, K, GELEM), jnp.float32),
            jax.ShapeDtypeStruct((BSZ, 1, L), jnp.int32),
        ],
        scratch_shapes=[pltpu.VMEM((BEAM, VPAD), jnp.float32)],
    )


def _sc_body(cand_hbm, gid_hbm, vals_hbm, flats_hbm,
             cbuf, gbuf, ovst, ofst, sem0):
    wid = lax.axis_index("c") * 16 + lax.axis_index("s")
    iota = lax.iota(jnp.int32, L)
    negv = jnp.full((L,), NEG, dtype=jnp.float32)
    zerov = jnp.zeros((L,), dtype=jnp.int32)

    def row_body(r, _):
        row = wid * ROWS_PER_W + r
        c1 = pltpu.async_copy(cand_hbm.at[row], cbuf, sem0)
        pltpu.sync_copy(gid_hbm.at[row], gbuf)
        c1.wait()

        gv = gbuf[...]
        skeys, svals = plsc.sort_key_val(gv, iota)
        ov = jnp.zeros((L,), dtype=jnp.float32)
        of = zerov

        def lanemax(pk):
            def lm_step(v, lm):
                return jnp.maximum(lm, cbuf[pl.ds(pk * GELEM + v * L, L)])
            return lax.fori_loop(0, GELEM // L, lm_step, negv)

        S = negv
        for kk in range(K):
            pk = jnp.min(jnp.where(iota == kk, svals, IMAX))
            S = jnp.where(iota == kk, jnp.max(lanemax(pk)), S)

        for k in range(K):
            gmax = jnp.max(S)
            kstar = jnp.min(jnp.where(S == gmax, iota, IMAX))
            sid = jnp.min(jnp.where(iota == kstar, skeys, IMAX))
            pk = jnp.min(jnp.where(iota == kstar, svals, IMAX))
            beam = sid // GP_BEAM
            base = beam * VOCAB + (sid - beam * GP_BEAM) * GELEM

            def minix_step(v, mi, pk=pk, base=base, gmax=gmax):
                x = cbuf[pl.ds(pk * GELEM + v * L, L)]
                ixv = base + v * L + iota
                return jnp.minimum(mi, jnp.where(x == gmax, ixv, IMAX))

            minIX = lax.fori_loop(0, GELEM // L, minix_step,
                                  jnp.full((L,), IMAX, dtype=jnp.int32))
            istar = jnp.min(minIX)
            pstar = pk * GELEM + (istar - base)
            plsc.store_scatter(cbuf, [jnp.broadcast_to(pstar, (L,))], negv,
                               mask=iota == 0)
            S = jnp.where(iota == kstar, jnp.max(lanemax(pk)), S)
            ov = jnp.where(iota == k, gmax, ov)
            of = jnp.where(iota == k, istar, of)

        ovst[...] = ov
        ofst[...] = of
        pltpu.sync_copy(ovst, vals_hbm.at[row])
        pltpu.sync_copy(ofst, flats_hbm.at[row])
        return 0

    lax.fori_loop(0, ROWS_PER_W, row_body, 0)


_sc_stage = pl.kernel(
    _sc_body,
    out_type=[
        jax.ShapeDtypeStruct((BSZ, L), jnp.float32),
        jax.ShapeDtypeStruct((BSZ, L), jnp.int32),
    ],
    mesh=plsc.VectorSubcoreMesh(core_axis_name="c", subcore_axis_name="s",
                                num_cores=2, num_subcores=16),
    compiler_params=pltpu.CompilerParams(use_tc_tiling_on_sc=False,
                                         needs_layout_passes=False),
    scratch_types=[
        pltpu.VMEM((CAND,), jnp.float32),
        pltpu.VMEM((L,), jnp.int32),
        pltpu.VMEM((L,), jnp.float32),
        pltpu.VMEM((L,), jnp.int32),
        pltpu.SemaphoreType.DMA,
    ],
)


def kernel(step, lprobs, scores):
    global _tc_stage
    if _tc_stage is None:
        _tc_stage = _make_tc_stage()
    bsz, beam, vocab = lprobs.shape
    bias = jnp.take(scores, step - 1, axis=2)
    cand, gids = _tc_stage(lprobs, bias[:, :, None])
    vals, flats = _sc_stage(cand.reshape(bsz, CAND),
                            gids.reshape(bsz, L))
    vals = vals[:, :K]
    flats = flats[:, :K]
    return (vals, flats % vocab, flats // vocab)
